# Initial kernel scaffold; baseline (speedup 1.0000x reference)
import jax
import jax.numpy as jnp
from jax.experimental import pallas as pl


def kernel(q, knots_x, knots_y, cell):
    raise NotImplementedError("write your pallas kernel here")


# speedup vs baseline: 94.2607x; 94.2607x over previous
import functools

import jax
import jax.numpy as jnp
from jax import lax
from jax.experimental import pallas as pl
from jax.experimental.pallas import tpu as pltpu
from jax.experimental.pallas import tpu_sc as plsc

N_ATOMS = 2048
NUM_CORES = 2
NUM_SUBCORES = 16
NW = NUM_CORES * NUM_SUBCORES
IPW = N_ATOMS // NW
NIV = IPW // 16
CUTOFF = 2.3
MAGIC = 0x5F3759DF


def _vgather(vec, idx):
    return lax.gather(
        vec, idx[:, None],
        dimension_numbers=lax.GatherDimensionNumbers(
            offset_dims=(), collapsed_slice_dims=(0,), start_index_map=(0,)),
        slice_sizes=(1,),
        mode=lax.GatherScatterMode.PROMISE_IN_BOUNDS)


def _sc_body(qx_h, qy_h, qz_h, coef_h, cst_h, fx_h, fy_h, fz_h,
             qx_v, qy_v, qz_v, coef_v, cst_v, ob_v):
    wid = lax.axis_index("s") * NUM_CORES + lax.axis_index("c")
    base = wid * IPW

    pltpu.sync_copy(qx_h, qx_v)
    pltpu.sync_copy(qy_h, qy_v)
    pltpu.sync_copy(qz_h, qz_v)
    pltpu.sync_copy(coef_h, coef_v)
    pltpu.sync_copy(cst_h, cst_v)

    lx = cst_v[0]
    ly = cst_v[1]
    lz = cst_v[2]
    x0s = cst_v[3]
    invh = cst_v[4]
    hlx = 0.5 * lx
    hly = 0.5 * ly
    hlz = 0.5 * lz

    qxi = [qx_v[pl.ds(base + iv * 16, 16)] for iv in range(NIV)]
    qyi = [qy_v[pl.ds(base + iv * 16, 16)] for iv in range(NIV)]
    qzi = [qz_v[pl.ds(base + iv * 16, 16)] for iv in range(NIV)]

    zeros = jnp.zeros((16,), jnp.float32)

    c0t = coef_v[0]
    c1t = coef_v[1]
    c2t = coef_v[2]
    c3t = coef_v[3]
    lane_splats = [jnp.full((16,), k, jnp.int32) for k in range(16)]

    def jchunk(jc, acc):
        qxc = qx_v[pl.ds(jc * 16, 16)]
        qyc = qy_v[pl.ds(jc * 16, 16)]
        qzc = qz_v[pl.ds(jc * 16, 16)]
        acc = list(acc)
        for k in range(16):
            qxj = _vgather(qxc, lane_splats[k])
            qyj = _vgather(qyc, lane_splats[k])
            qzj = _vgather(qzc, lane_splats[k])
            for iv in range(NIV):
                dx = qxi[iv] - qxj
                dy = qyi[iv] - qyj
                dz = qzi[iv] - qzj
                dx = dx - jnp.where(dx > hlx, lx, jnp.where(dx < -hlx, -lx, zeros))
                dy = dy - jnp.where(dy > hly, ly, jnp.where(dy < -hly, -ly, zeros))
                dz = dz - jnp.where(dz > hlz, lz, jnp.where(dz < -hlz, -lz, zeros))
                r2 = dx * dx + dy * dy + dz * dz
                bits = lax.bitcast_convert_type(r2, jnp.int32)
                y = lax.bitcast_convert_type(MAGIC - (bits >> 1), jnp.float32)
                hs = 0.5 * r2
                y = y * (1.5 - hs * y * y)
                y = y * (1.5 - hs * y * y)
                y = y * (1.5 - hs * y * y)
                r = r2 * y
                ii = ((r - x0s) * invh).astype(jnp.int32)
                ii = jnp.minimum(jnp.maximum(ii, 0), 14)
                f = ((_vgather(c3t, ii) * r + _vgather(c2t, ii)) * r
                     + _vgather(c1t, ii)) * r + _vgather(c0t, ii)
                valid = (r < CUTOFF) & (r2 > 0.0)
                g = jnp.where(valid, f * y, zeros)
                ax, ay, az = acc[iv]
                acc[iv] = (ax + g * dx, ay + g * dy, az + g * dz)
        return tuple(acc)

    acc = tuple((zeros, zeros, zeros) for _ in range(NIV))
    acc = lax.fori_loop(0, N_ATOMS // 16, jchunk, acc)

    for iv in range(NIV):
        ax, ay, az = acc[iv]
        ob_v[0, pl.ds(iv * 16, 16)] = ax
        ob_v[1, pl.ds(iv * 16, 16)] = ay
        ob_v[2, pl.ds(iv * 16, 16)] = az

    pltpu.sync_copy(ob_v.at[0], fx_h.at[pl.ds(base, IPW)])
    pltpu.sync_copy(ob_v.at[1], fy_h.at[pl.ds(base, IPW)])
    pltpu.sync_copy(ob_v.at[2], fz_h.at[pl.ds(base, IPW)])


@jax.jit
def kernel(q, knots_x, knots_y, cell):
    f32 = jnp.float32
    x = knots_x.astype(f32)
    yk = knots_y.astype(f32)
    h = x[1:] - x[:-1]
    one = jnp.array([1.0], f32)
    zero = jnp.array([0.0], f32)
    main = jnp.concatenate([one, 2.0 * (h[:-1] + h[1:]), one])
    sub = jnp.concatenate([h[:-1], zero])
    sup = jnp.concatenate([zero, h[1:]])
    A = jnp.diag(main) + jnp.diag(sub, -1) + jnp.diag(sup, 1)
    b = jnp.concatenate(
        [zero, 6.0 * ((yk[2:] - yk[1:-1]) / h[1:] - (yk[1:-1] - yk[:-2]) / h[:-1]), zero])
    M = jnp.linalg.solve(A, b)

    x0 = x[:-1]; x1 = x[1:]
    M0 = M[:-1]; M1 = M[1:]
    y0 = yk[:-1]; y1 = yk[1:]
    A3 = M0 / (6.0 * h); B3 = M1 / (6.0 * h)
    C = y0 / h - M0 * h / 6.0
    D = y1 / h - M1 * h / 6.0
    c0 = A3 * x1**3 - B3 * x0**3 + C * x1 - D * x0
    c1 = -3.0 * A3 * x1**2 + 3.0 * B3 * x0**2 - C + D
    c2 = 3.0 * A3 * x1 - 3.0 * B3 * x0
    c3 = -A3 + B3
    pad = lambda c: jnp.concatenate([c, c[-1:]]).astype(f32)
    coef = jnp.stack([pad(c0), pad(c1), pad(c2), pad(c3)])

    invh = (1.0 / h[0]).astype(f32)
    cst = jnp.stack([
        jnp.full((16,), cell[0], f32),
        jnp.full((16,), cell[1], f32),
        jnp.full((16,), cell[2], f32),
        jnp.full((16,), x[0], f32),
        jnp.full((16,), invh, f32),
    ])

    qx = q[:, 0].astype(f32)
    qy = q[:, 1].astype(f32)
    qz = q[:, 2].astype(f32)

    sc = pl.kernel(
        _sc_body,
        out_type=[jax.ShapeDtypeStruct((N_ATOMS,), f32)] * 3,
        mesh=plsc.VectorSubcoreMesh(core_axis_name="c", subcore_axis_name="s"),
        scratch_types=[
            pltpu.VMEM((N_ATOMS,), f32),
            pltpu.VMEM((N_ATOMS,), f32),
            pltpu.VMEM((N_ATOMS,), f32),
            pltpu.VMEM((4, 16), f32),
            pltpu.VMEM((5, 16), f32),
            pltpu.VMEM((3, IPW), f32),
        ],
    )
    fx, fy, fz = sc(qx, qy, qz, coef, cst)
    return jnp.stack([fx, fy, fz], axis=1)


# speedup vs baseline: 125.8913x; 1.3356x over previous
import functools

import jax
import jax.numpy as jnp
from jax import lax
from jax.experimental import pallas as pl
from jax.experimental.pallas import tpu as pltpu
from jax.experimental.pallas import tpu_sc as plsc

N_ATOMS = 2048
NUM_CORES = 2
NUM_SUBCORES = 16
NW = NUM_CORES * NUM_SUBCORES
IPW = N_ATOMS // NW
NCHUNK = N_ATOMS // 16
CUTOFF = 2.3
MAGIC = 0x5F3759DF
UNROLL = 4


def _vgather(vec, idx):
    return lax.gather(
        vec, idx[:, None],
        dimension_numbers=lax.GatherDimensionNumbers(
            offset_dims=(), collapsed_slice_dims=(0,), start_index_map=(0,)),
        slice_sizes=(1,),
        mode=lax.GatherScatterMode.PROMISE_IN_BOUNDS)


def _pairs_body(qx_h, qy_h, qz_h, coef_h, cst_h, part_h,
                qx_v, qy_v, qz_v, coef_v, cst_v, fxa_v, fya_v, fza_v):
    wid = lax.axis_index("s") * NUM_CORES + lax.axis_index("c")

    pltpu.sync_copy(qx_h, qx_v)
    pltpu.sync_copy(qy_h, qy_v)
    pltpu.sync_copy(qz_h, qz_v)
    pltpu.sync_copy(coef_h, coef_v)
    pltpu.sync_copy(cst_h, cst_v)

    lx = cst_v[0]
    ly = cst_v[1]
    lz = cst_v[2]
    x0s = cst_v[3]
    invh = cst_v[4]
    hlx = 0.5 * lx
    hly = 0.5 * ly
    hlz = 0.5 * lz

    c0t = coef_v[0]
    c1t = coef_v[1]
    c2t = coef_v[2]
    c3t = coef_v[3]

    zeros = jnp.zeros((16,), jnp.float32)
    lanes = lax.iota(jnp.int32, 16)

    def zstep(c, carry):
        fxa_v[pl.ds(c * 16, 16)] = zeros
        fya_v[pl.ds(c * 16, 16)] = zeros
        fza_v[pl.ds(c * 16, 16)] = zeros
        return carry

    lax.fori_loop(0, NCHUNK, zstep, 0)

    def istep(t, carry):
        i = wid + NW * t
        ic = i // 16
        il = i - ic * 16
        qxc = qx_v[pl.ds(ic * 16, 16)]
        qyc = qy_v[pl.ds(ic * 16, 16)]
        qzc = qz_v[pl.ds(ic * 16, 16)]
        ilv = jnp.full((16,), il, jnp.int32)
        qxi = _vgather(qxc, ilv)
        qyi = _vgather(qyc, ilv)
        qzi = _vgather(qzc, ilv)
        jc0 = (ic // UNROLL) * UNROLL

        def jstep(s, acc):
            ax, ay, az = acc
            for u in range(UNROLL):
                jc = jc0 + s * UNROLL + u
                jb = jc * 16
                dx = qxi - qx_v[pl.ds(jb, 16)]
                dy = qyi - qy_v[pl.ds(jb, 16)]
                dz = qzi - qz_v[pl.ds(jb, 16)]
                dx = dx - jnp.where(dx > hlx, lx, jnp.where(dx < -hlx, -lx, zeros))
                dy = dy - jnp.where(dy > hly, ly, jnp.where(dy < -hly, -ly, zeros))
                dz = dz - jnp.where(dz > hlz, lz, jnp.where(dz < -hlz, -lz, zeros))
                r2 = dx * dx + dy * dy + dz * dz
                bits = lax.bitcast_convert_type(r2, jnp.int32)
                y = lax.bitcast_convert_type(MAGIC - (bits >> 1), jnp.float32)
                hs = 0.5 * r2
                y = y * (1.5 - hs * y * y)
                y = y * (1.5 - hs * y * y)
                y = y * (1.5 - hs * y * y)
                r = r2 * y
                ii = ((r - x0s) * invh).astype(jnp.int32)
                ii = jnp.minimum(jnp.maximum(ii, 0), 14)
                f = ((_vgather(c3t, ii) * r + _vgather(c2t, ii)) * r
                     + _vgather(c1t, ii)) * r + _vgather(c0t, ii)
                jv = jb + lanes
                valid = (r < CUTOFF) & (r2 > 0.0) & (jv > i)
                g = jnp.where(valid, f * y, zeros)
                gx = g * dx
                gy = g * dy
                gz = g * dz
                ax = ax + gx
                ay = ay + gy
                az = az + gz
                plsc.addupdate(fxa_v.at[pl.ds(jb, 16)], zeros - gx)
                plsc.addupdate(fya_v.at[pl.ds(jb, 16)], zeros - gy)
                plsc.addupdate(fza_v.at[pl.ds(jb, 16)], zeros - gz)
            return (ax, ay, az)

        trip = (NCHUNK - jc0) // UNROLL
        ax, ay, az = lax.fori_loop(0, trip, jstep, (zeros, zeros, zeros))

        for sh in (8, 4, 2, 1):
            perm = lanes ^ sh
            ax = ax + _vgather(ax, perm)
            ay = ay + _vgather(ay, perm)
            az = az + _vgather(az, perm)
        lm = lanes == il
        ib = ic * 16
        plsc.addupdate(fxa_v.at[pl.ds(ib, 16)], jnp.where(lm, ax, zeros))
        plsc.addupdate(fya_v.at[pl.ds(ib, 16)], jnp.where(lm, ay, zeros))
        plsc.addupdate(fza_v.at[pl.ds(ib, 16)], jnp.where(lm, az, zeros))
        return carry

    lax.fori_loop(0, IPW, istep, 0)

    pltpu.sync_copy(fxa_v, part_h.at[wid * 3 + 0])
    pltpu.sync_copy(fya_v, part_h.at[wid * 3 + 1])
    pltpu.sync_copy(fza_v, part_h.at[wid * 3 + 2])


def _sum_body(part_h, fx_h, fy_h, fz_h, blk_v, ob_v):
    wid = lax.axis_index("s") * NUM_CORES + lax.axis_index("c")
    base = wid * 128

    @pl.when(wid < N_ATOMS // 128)
    def _():
        pltpu.sync_copy(part_h.at[:, pl.ds(base, 128)], blk_v)
        for comp, out_h in ((0, fx_h), (1, fy_h), (2, fz_h)):
            for v in range(128 // 16):
                acc = blk_v[comp, pl.ds(v * 16, 16)]
                for w in range(1, NW):
                    acc = acc + blk_v[w * 3 + comp, pl.ds(v * 16, 16)]
                ob_v[comp, pl.ds(v * 16, 16)] = acc
            pltpu.sync_copy(ob_v.at[comp], out_h.at[pl.ds(base, 128)])


@jax.jit
def kernel(q, knots_x, knots_y, cell):
    f32 = jnp.float32
    x = knots_x.astype(f32)
    yk = knots_y.astype(f32)
    h = x[1:] - x[:-1]
    one = jnp.array([1.0], f32)
    zero = jnp.array([0.0], f32)
    main = jnp.concatenate([one, 2.0 * (h[:-1] + h[1:]), one])
    sub = jnp.concatenate([h[:-1], zero])
    sup = jnp.concatenate([zero, h[1:]])
    A = jnp.diag(main) + jnp.diag(sub, -1) + jnp.diag(sup, 1)
    b = jnp.concatenate(
        [zero, 6.0 * ((yk[2:] - yk[1:-1]) / h[1:] - (yk[1:-1] - yk[:-2]) / h[:-1]), zero])
    M = jnp.linalg.solve(A, b)

    x0 = x[:-1]; x1 = x[1:]
    M0 = M[:-1]; M1 = M[1:]
    y0 = yk[:-1]; y1 = yk[1:]
    A3 = M0 / (6.0 * h); B3 = M1 / (6.0 * h)
    C = y0 / h - M0 * h / 6.0
    D = y1 / h - M1 * h / 6.0
    c0 = A3 * x1**3 - B3 * x0**3 + C * x1 - D * x0
    c1 = -3.0 * A3 * x1**2 + 3.0 * B3 * x0**2 - C + D
    c2 = 3.0 * A3 * x1 - 3.0 * B3 * x0
    c3 = -A3 + B3
    pad = lambda c: jnp.concatenate([c, c[-1:]]).astype(f32)
    coef = jnp.stack([pad(c0), pad(c1), pad(c2), pad(c3)])

    invh = (1.0 / h[0]).astype(f32)
    cst = jnp.stack([
        jnp.full((16,), cell[0], f32),
        jnp.full((16,), cell[1], f32),
        jnp.full((16,), cell[2], f32),
        jnp.full((16,), x[0], f32),
        jnp.full((16,), invh, f32),
    ])

    qx = q[:, 0].astype(f32)
    qy = q[:, 1].astype(f32)
    qz = q[:, 2].astype(f32)

    mesh = plsc.VectorSubcoreMesh(core_axis_name="c", subcore_axis_name="s")

    pairs = pl.kernel(
        _pairs_body,
        out_type=jax.ShapeDtypeStruct((NW * 3, N_ATOMS), f32),
        mesh=mesh,
        scratch_types=[
            pltpu.VMEM((N_ATOMS,), f32),
            pltpu.VMEM((N_ATOMS,), f32),
            pltpu.VMEM((N_ATOMS,), f32),
            pltpu.VMEM((4, 16), f32),
            pltpu.VMEM((5, 16), f32),
            pltpu.VMEM((N_ATOMS,), f32),
            pltpu.VMEM((N_ATOMS,), f32),
            pltpu.VMEM((N_ATOMS,), f32),
        ],
    )
    part = pairs(qx, qy, qz, coef, cst)

    combine = pl.kernel(
        _sum_body,
        out_type=[jax.ShapeDtypeStruct((N_ATOMS,), f32)] * 3,
        mesh=mesh,
        scratch_types=[
            pltpu.VMEM((NW * 3, 128), f32),
            pltpu.VMEM((3, 128), f32),
        ],
    )
    fx, fy, fz = combine(part)
    return jnp.stack([fx, fy, fz], axis=1)
